# baseline (speedup 1.0000x reference)
import functools

import jax
import jax.numpy as jnp
from jax import lax
from jax.experimental import pallas as pl
from jax.experimental.pallas import tpu as pltpu
from jax.experimental.pallas import tpu_sc as plsc

N = 10000
E = 320000
D_IN = 128
D = 64
NC, NS = 2, 16
CHUNK = 128
EPW = 160
EPAD = NS * EPW * CHUNK
NRP = 10112
NPS = NRP // NS
NPAD = NRP + 16

def _spmm_body(a_hbm, srcg_hbm, dst_hbm, zeros_hbm, out_hbm,
               src_v, dst_v, buf0, buf1, buf2, buf3, s_sh,
               gs0, gs1, gs2, gs3, ss0, ss1, ss2, ss3):
    c = lax.axis_index("c")
    s = lax.axis_index("s")
    r0 = s * NPS
    pltpu.sync_copy(zeros_hbm.at[pl.ds(r0, NPS)], s_sh.at[pl.ds(r0, NPS)])
    pltpu.sync_copy(srcg_hbm.at[c, s], src_v)
    pltpu.sync_copy(dst_hbm.at[c, s], dst_v)
    plsc.subcore_barrier()
    bufs = (buf0, buf1, buf2, buf3)
    gs = (gs0, gs1, gs2, gs3)
    ss = (ss0, ss1, ss2, ss3)

    def gat(j, b):
        pltpu.async_copy(a_hbm.at[src_v.at[j]], bufs[b], gs[b])

    def gat_wait(j, b):
        pltpu.make_async_copy(a_hbm.at[src_v.at[j]], bufs[b], gs[b]).wait()

    def sca(j, b):
        pltpu.async_copy(bufs[b], s_sh.at[dst_v.at[j]], ss[b], add=True)

    def sca_wait(j, b):
        pltpu.make_async_copy(bufs[b], s_sh.at[dst_v.at[j]], ss[b]).wait()

    gat(0, 0)
    gat(1, 1)
    for j in range(2):
        gat_wait(j, j)
        sca(j, j)
        gat(j + 2, j + 2)

    def grp(g, carry):
        for i in range(4):
            k = 4 * g + 2 + i
            b = (2 + i) % 4
            gat_wait(k, b)
            sca(k, b)
            sca_wait(k - 2, i)
            gat(k + 2, i)
        return carry

    lax.fori_loop(0, (EPW - 4) // 4, grp, 0)
    for k in (EPW - 2, EPW - 1):
        b = k % 4
        gat_wait(k, b)
        sca(k, b)
        sca_wait(k - 2, (k - 2) % 4)
    for k in (EPW - 2, EPW - 1):
        sca_wait(k, k % 4)
    plsc.subcore_barrier()
    pltpu.sync_copy(s_sh.at[pl.ds(r0, NPS)], out_hbm.at[c, pl.ds(r0, NPS)])


def _degk_body(dst_hbm, zeros_hbm, out_hbm, dst_v, ones_v, d_sh):
    c = lax.axis_index("c")
    s = lax.axis_index("s")
    r0 = s * NPS
    pltpu.sync_copy(zeros_hbm.at[pl.ds(r0, NPS)], d_sh.at[pl.ds(r0, NPS)])
    pltpu.sync_copy(dst_hbm.at[c, s], dst_v)

    def fill(i, carry):
        for j in range(D // 16):
            ones_v[i, pl.ds(16 * j, 16)] = jnp.ones((16,), jnp.float32)
        return carry

    lax.fori_loop(0, CHUNK, fill, 0)
    plsc.subcore_barrier()

    def body(j, carry):
        pltpu.sync_copy(ones_v, d_sh.at[dst_v.at[j]], add=True)
        return carry

    lax.fori_loop(0, EPW, body, 0)
    plsc.subcore_barrier()
    pltpu.sync_copy(d_sh.at[pl.ds(r0, NPS)], out_hbm.at[c, pl.ds(r0, NPS)])


_SC_KERNELS = None


def _get_sc_kernels():
    global _SC_KERNELS
    if _SC_KERNELS is None:
        mesh = plsc.VectorSubcoreMesh(core_axis_name="c", subcore_axis_name="s",
                                      num_cores=NC, num_subcores=NS)
        spmm = pl.kernel(
            _spmm_body,
            out_type=jax.ShapeDtypeStruct((NC, NRP, D), jnp.float32),
            mesh=mesh,
            compiler_params=pltpu.CompilerParams(use_tc_tiling_on_sc=False),
            scratch_types=(
                [pltpu.VMEM((EPW, CHUNK), jnp.int32),
                 pltpu.VMEM((EPW, CHUNK), jnp.int32)]
                + [pltpu.VMEM((CHUNK, D), jnp.float32)] * 4
                + [pltpu.VMEM_SHARED((NPAD, D), jnp.float32)]
                + [pltpu.SemaphoreType.DMA] * 8),
        )
        degk = pl.kernel(
            _degk_body,
            out_type=jax.ShapeDtypeStruct((NC, NRP, D), jnp.float32),
            mesh=mesh,
            compiler_params=pltpu.CompilerParams(use_tc_tiling_on_sc=False),
            scratch_types=[
                pltpu.VMEM((EPW, CHUNK), jnp.int32),
                pltpu.VMEM((CHUNK, D), jnp.float32),
                pltpu.VMEM_SHARED((NPAD, D), jnp.float32),
            ],
        )
        _SC_KERNELS = (spmm, degk)
    return _SC_KERNELS


def _spmm(a_flat, srcg, dstp, z64):
    return _get_sc_kernels()[0](a_flat, srcg, dstp, z64)


def _degk(dstp, z64):
    return _get_sc_kernels()[1](dstp, z64)


def _dotp(x, w, prec):
    return jnp.dot(x, w, preferred_element_type=jnp.float32, precision=prec)


def _dot3(x, w):
    return jnp.dot(x, w, preferred_element_type=jnp.float32)


def _dot3t(x, w):
    return lax.dot_general(x, w, (((0,), (0,)), ((), ())),
                           preferred_element_type=jnp.float32,
                           precision=lax.Precision.HIGHEST)


def _mats_body_any(prec, x_ref, deg_ref, w1, b1, w2, b2, w3, b3,
                   a_ref, d_ref):
    h = x_ref[0]
    deg = deg_ref[0]
    a_ref[0] = _dotp(h, w1[...], prec) + b1[...]
    d_ref[0] = (_dotp(h, w3[...], prec) + b3[...]
                - deg * (_dotp(h, w2[...], prec) + b2[...]))


def _bn_relu(h2, gam, bet):
    mean = jnp.mean(h2, axis=0, keepdims=True)
    var = jnp.mean((h2 - mean) ** 2, axis=0, keepdims=True)
    return jnp.maximum((h2 - mean) / jnp.sqrt(var + 1e-5) * gam + bet, 0.0)


def _bnk_body(s_ref, dp_ref, gam, bet, h_ref):
    h_ref[0] = _bn_relu(s_ref[0][:N] + dp_ref[0], gam[...], bet[...])


def _pool_body(h_ref, batch_ref, gw1, gb1, gw2, gb2, emb_ref):
    h = h_ref[0]
    g1 = jnp.maximum(
        _dot3(h, gw1[...]) + gb1[...], 0.0)
    gate = jnp.maximum(
        _dot3(g1, gw2[...]) + gb2[...], 0.0)
    b = batch_ref[0, 0]
    gids = lax.broadcasted_iota(jnp.int32, (N, 64), 1)
    m = b[:, None] == gids
    mf = m.astype(jnp.float32)
    scores = jnp.where(m, gate, -jnp.inf)
    gmax = jnp.max(scores, axis=0)
    gmax = jnp.where(jnp.isfinite(gmax), gmax, 0.0)
    gmax_n = jnp.max(jnp.where(m, gmax[None, :], -jnp.inf), axis=1,
                     keepdims=True)
    e = jnp.exp(gate - gmax_n)
    esum = _dot3t(mf, e)
    esum_n = jnp.sum(jnp.where(m, esum[:, 0][None, :], 0.0), axis=1,
                     keepdims=True)
    attn = e / (esum_n + 1e-16)
    emb_ref[0] = _dot3t(mf, attn * h)


def _head_body(emb_ref, ow0, ob0, ow1, ob1, ow2, ob2, ow3, ob3, out_ref):
    h = jnp.abs(emb_ref[0] - emb_ref[1])
    for w, b in ((ow0, ob0), (ow1, ob1), (ow2, ob2)):
        h = jnp.maximum(_dot3(h, w[...]) + b[...], 0.0)
    out_ref[...] = (_dot3(h, ow3[...]) + ob3[...])


def _tower_spec(dim):
    return pl.BlockSpec((1, N, dim), lambda t: (t, 0, 0))


def _ptower_spec(dim):
    return pl.BlockSpec((1, NRP, dim), lambda t: (t, 0, 0))


def _full_spec(shape):
    nd = len(shape)
    return pl.BlockSpec(shape, lambda t: (0,) * nd)


RB = 2000


def _mats_call(x, deg, w1, b1, w2, b2, w3, b3):
    din = x.shape[-1]

    def rspec(dim):
        return pl.BlockSpec((1, RB, dim), lambda t, r: (t, r, 0))

    def wspec(shape):
        nd = len(shape)
        return pl.BlockSpec(shape, lambda t, r: (0,) * nd)

    prec = lax.Precision.DEFAULT
    return pl.pallas_call(
        functools.partial(_mats_body_any, prec),
        grid=(2, N // RB),
        in_specs=[rspec(din), rspec(D),
                  wspec((din, D)), wspec((1, D)),
                  wspec((din, D)), wspec((1, D)),
                  wspec((din, D)), wspec((1, D))],
        out_specs=[rspec(D), rspec(D)],
        out_shape=[jax.ShapeDtypeStruct((2, N, D), jnp.float32),
                   jax.ShapeDtypeStruct((2, N, D), jnp.float32)],
        compiler_params=pltpu.CompilerParams(
            dimension_semantics=("arbitrary", "arbitrary")),
    )(x, deg, w1, b1, w2, b2, w3, b3)


def _bnk_call(s, d, gam, bet):
    return pl.pallas_call(
        _bnk_body,
        grid=(2,),
        in_specs=[_ptower_spec(D), _tower_spec(D),
                  _full_spec((1, D)), _full_spec((1, D))],
        out_specs=_tower_spec(D),
        out_shape=jax.ShapeDtypeStruct((2, N, D), jnp.float32),
        compiler_params=pltpu.CompilerParams(
            dimension_semantics=("arbitrary",)),
    )(s, d, gam, bet)


def _pool_call(h, batch2, gw1, gb1, gw2, gb2):
    return pl.pallas_call(
        _pool_body,
        grid=(2,),
        in_specs=[_tower_spec(D),
                  pl.BlockSpec((1, 1, N), lambda t: (t, 0, 0)),
                  _full_spec((D, 32)), _full_spec((1, 32)),
                  _full_spec((32, 1)), _full_spec((1, 1))],
        out_specs=pl.BlockSpec((1, 64, D), lambda t: (t, 0, 0)),
        out_shape=jax.ShapeDtypeStruct((2, 64, D), jnp.float32),
        compiler_params=pltpu.CompilerParams(
            dimension_semantics=("arbitrary",)),
    )(h, batch2, gw1, gb1, gw2, gb2)


def _head_call(emb, *outp):
    return pl.pallas_call(
        _head_body,
        out_shape=jax.ShapeDtypeStruct((64, 1), jnp.float32),
    )(emb, *outp)


def kernel(x_s, x_t, params, edge_index_s, batch_s, edge_index_t, batch_t):
    i32 = jnp.int32
    f32 = jnp.float32
    srcs, dsts = [], []
    for t, ei in enumerate((edge_index_s, edge_index_t)):
        pad = EPAD - E
        src = jnp.concatenate([ei[0] + t * N, jnp.full((pad,), t * N, i32)])
        dst = jnp.concatenate([ei[1], jnp.full((pad,), NRP, i32)])
        srcs.append(src.reshape(NS, EPW, CHUNK))
        dsts.append(dst.reshape(NS, EPW, CHUNK))
    srcg = jnp.stack(srcs)
    dstp = jnp.stack(dsts)
    z64 = jnp.zeros((NRP, D), f32)
    deg = _degk(dstp, z64)
    x = jnp.stack([x_s, x_t])
    batch2 = jnp.stack([batch_s, batch_t])[:, None, :]

    def wb(l):
        return (params["conv%d_W1" % l], params["conv%d_b1" % l][None, :],
                params["conv%d_W2" % l], params["conv%d_b2" % l][None, :],
                params["conv%d_W3" % l], params["conv%d_b3" % l][None, :])

    a, d = _mats_call(x, deg, *wb(0))
    for l in range(5):
        s = _spmm(a.reshape(NC * N, D), srcg, dstp, z64)
        h = _bnk_call(s, d, params["conv%d_gamma" % l][None, :],
                      params["conv%d_beta" % l][None, :])
        a, d = _mats_call(h, deg, *wb(l + 1))
    s = _spmm(a.reshape(NC * N, D), srcg, dstp, z64)
    h = _bnk_call(s, d, params["conv5_gamma"][None, :],
                  params["conv5_beta"][None, :])
    emb = _pool_call(
        h, batch2,
        params["gate_W1"], params["gate_b1"][None, :],
        params["gate_W2"], params["gate_b2"][None, :])
    return _head_call(
        emb,
        params["out_W0"], params["out_b0"][None, :],
        params["out_W1"], params["out_b1"][None, :],
        params["out_W2"], params["out_b2"][None, :],
        params["out_W3"], params["out_b3"][None, :])
